# Initial kernel scaffold; baseline (speedup 1.0000x reference)
import functools
import jax
import jax.numpy as jnp
from jax import lax
from jax.experimental import pallas as pl
from jax.experimental.pallas import tpu as pltpu
from jax.experimental.pallas import tpu_sc as plsc

NC = 2
NS = 16
NW = NC * NS

IDX_ROW = 128
ROWS_PER_STEP = 1024
GATHERS = ROWS_PER_STEP // IDX_ROW


def _make_gather(R, D):
    assert R % (NW * ROWS_PER_STEP) == 0
    rows_per_w = R // NW
    steps = rows_per_w // ROWS_PER_STEP
    idx_rows_per_w = rows_per_w // IDX_ROW

    mesh = plsc.VectorSubcoreMesh(core_axis_name="c", subcore_axis_name="s")

    @functools.partial(
        pl.kernel,
        mesh=mesh,
        out_type=jax.ShapeDtypeStruct((R, D), jnp.float32),
        scratch_types=[
            pltpu.VMEM((GATHERS, IDX_ROW), jnp.int32),
            pltpu.VMEM((ROWS_PER_STEP, D), jnp.float32),
            pltpu.SemaphoreType.DMA,
        ],
    )
    def gather_kernel(idx_hbm, table_hbm, out_hbm, idx_v, rows_v, sem):
        wid = lax.axis_index("s") * NC + lax.axis_index("c")
        idx_row0 = wid * idx_rows_per_w
        out_row0 = wid * rows_per_w

        @pl.loop(0, steps)
        def step(g):
            pltpu.sync_copy(idx_hbm.at[pl.ds(idx_row0 + g * GATHERS, GATHERS)],
                            idx_v)
            copies = [
                pltpu.async_copy(
                    table_hbm.at[idx_v.at[j]],
                    rows_v.at[pl.ds(j * IDX_ROW, IDX_ROW)],
                    sem,
                )
                for j in range(GATHERS)
            ]
            for c in copies:
                c.wait()
            pltpu.sync_copy(
                rows_v,
                out_hbm.at[pl.ds(out_row0 + g * ROWS_PER_STEP, ROWS_PER_STEP)])

    return gather_kernel


def kernel(x, Z):
    B, T = x.shape
    V, D = Z.shape
    R = B * T
    idx = x.reshape(R // IDX_ROW, IDX_ROW).astype(jnp.int32)
    out = _make_gather(R, D)(idx, Z)
    return out.reshape(B, T, D)


# speedup vs baseline: 4.8081x; 4.8081x over previous
import functools
import jax
import jax.numpy as jnp
from jax import lax
from jax.experimental import pallas as pl
from jax.experimental.pallas import tpu as pltpu
from jax.experimental.pallas import tpu_sc as plsc

NC = 2
NS = 16
NW = NC * NS

IDX_ROW = 128
ROWS_PER_STEP = 1024
GATHERS = ROWS_PER_STEP // IDX_ROW


def _make_gather(R, D):
    assert R % (NW * ROWS_PER_STEP) == 0
    rows_per_w = R // NW
    steps = rows_per_w // ROWS_PER_STEP
    idx_rows_per_w = rows_per_w // IDX_ROW

    mesh = plsc.VectorSubcoreMesh(core_axis_name="c", subcore_axis_name="s")

    @functools.partial(
        pl.kernel,
        mesh=mesh,
        out_type=jax.ShapeDtypeStruct((R, D), jnp.float32),
        scratch_types=[
            pltpu.VMEM((GATHERS, IDX_ROW), jnp.int32),
            pltpu.VMEM((ROWS_PER_STEP, D), jnp.float32),
            pltpu.SemaphoreType.DMA,
        ],
        compiler_params=pltpu.CompilerParams(use_tc_tiling_on_sc=False),
    )
    def gather_kernel(idx_hbm, table_hbm, out_hbm, idx_v, rows_v, sem):
        wid = lax.axis_index("s") * NC + lax.axis_index("c")
        idx_row0 = wid * idx_rows_per_w
        out_row0 = wid * rows_per_w

        @pl.loop(0, steps)
        def step(g):
            pltpu.sync_copy(idx_hbm.at[pl.ds(idx_row0 + g * GATHERS, GATHERS)],
                            idx_v)
            copies = [
                pltpu.async_copy(
                    table_hbm.at[idx_v.at[j]],
                    rows_v.at[pl.ds(j * IDX_ROW, IDX_ROW)],
                    sem,
                )
                for j in range(GATHERS)
            ]
            for c in copies:
                c.wait()
            pltpu.sync_copy(
                rows_v,
                out_hbm.at[pl.ds(out_row0 + g * ROWS_PER_STEP, ROWS_PER_STEP)])

    return gather_kernel


def kernel(x, Z):
    B, T = x.shape
    V, D = Z.shape
    R = B * T
    idx = x.reshape(R // IDX_ROW, IDX_ROW).astype(jnp.int32)
    out = _make_gather(R, D)(idx, Z)
    return out.reshape(B, T, D)


# speedup vs baseline: 5.4755x; 1.1388x over previous
import functools
import jax
import jax.numpy as jnp
from jax import lax
from jax.experimental import pallas as pl
from jax.experimental.pallas import tpu as pltpu
from jax.experimental.pallas import tpu_sc as plsc

NC = 2
NS = 16
NW = NC * NS

IDX_ROW = 128
ROWS_PER_BLOCK = 256
GPB = ROWS_PER_BLOCK // IDX_ROW
LANES = 128


def _make_gather(R):
    assert R % (NW * ROWS_PER_BLOCK) == 0
    rows_per_w = R // NW
    blocks = rows_per_w // ROWS_PER_BLOCK
    idx_rows_per_w = rows_per_w // IDX_ROW

    mesh = plsc.VectorSubcoreMesh(core_axis_name="c", subcore_axis_name="s")

    @functools.partial(
        pl.kernel,
        mesh=mesh,
        out_type=jax.ShapeDtypeStruct((R, LANES), jnp.float32),
        scratch_types=[
            pltpu.VMEM((2, GPB, IDX_ROW), jnp.int32),
            pltpu.VMEM((2, ROWS_PER_BLOCK, LANES), jnp.float32),
            pltpu.SemaphoreType.DMA,
            pltpu.SemaphoreType.DMA,
            pltpu.SemaphoreType.DMA,
        ],
    )
    def gather_kernel(idx_hbm, table_hbm, out_hbm, idx_v, rows_v, isem, gsem,
                      wsem):
        wid = lax.axis_index("s") * NC + lax.axis_index("c")
        idx_row0 = wid * idx_rows_per_w
        out_row0 = wid * rows_per_w

        pltpu.sync_copy(idx_hbm.at[pl.ds(idx_row0, GPB)], idx_v.at[0])

        @pl.loop(0, blocks)
        def block(b):
            buf = lax.rem(b, 2)

            @pl.when(b >= 2)
            def _():
                pltpu.make_async_copy(
                    rows_v.at[buf],
                    out_hbm.at[pl.ds(out_row0, ROWS_PER_BLOCK)],
                    wsem).wait()

            @pl.when(b >= 1)
            def _():
                pltpu.make_async_copy(
                    idx_hbm.at[pl.ds(idx_row0, GPB)], idx_v.at[0],
                    isem).wait()

            gathers = [
                pltpu.async_copy(
                    table_hbm.at[idx_v.at[buf, j]],
                    rows_v.at[buf, pl.ds(j * IDX_ROW, IDX_ROW)],
                    gsem,
                )
                for j in range(GPB)
            ]

            @pl.when(b + 1 < blocks)
            def _():
                pltpu.async_copy(
                    idx_hbm.at[pl.ds(idx_row0 + (b + 1) * GPB, GPB)],
                    idx_v.at[lax.rem(b + 1, 2)], isem)

            for g in gathers:
                g.wait()

            pltpu.async_copy(
                rows_v.at[buf],
                out_hbm.at[pl.ds(out_row0 + b * ROWS_PER_BLOCK,
                                 ROWS_PER_BLOCK)],
                wsem)

        @pl.loop(0, 2)
        def drain(i):
            pltpu.make_async_copy(
                rows_v.at[0],
                out_hbm.at[pl.ds(out_row0, ROWS_PER_BLOCK)],
                wsem).wait()

    return gather_kernel


def kernel(x, Z):
    B, T = x.shape
    V, D = Z.shape
    R = B * T
    idx = x.reshape(R // IDX_ROW, IDX_ROW).astype(jnp.int32)
    Zp = jnp.pad(Z, ((0, 0), (0, LANES - D)))
    out = _make_gather(R)(idx, Zp)
    return out[:, :D].reshape(B, T, D)
